# baseline (speedup 1.0000x reference)
import functools

import jax
import jax.numpy as jnp
from jax import lax
from jax.experimental import pallas as pl
from jax.experimental.pallas import tpu as pltpu
from jax.experimental.pallas import tpu_sc as plsc

_NC = 2
_NS = 16
_NW = _NC * _NS
_L = 16

_N = 16384 * 100
_CH = _N // _NW


def _body(x_hbm, tab_hbm, out_hbm, x_v, out_v, tab_v):
    wid = lax.axis_index("s") * _NC + lax.axis_index("c")
    base = wid * _CH
    pltpu.sync_copy(tab_hbm, tab_v)
    pltpu.sync_copy(x_hbm.at[pl.ds(base, _CH)], x_v)
    v = tab_v[...]
    t = 1.0 - 2.0 / (jnp.exp(2.0 * v) + 1.0)

    def step(i, carry):
        idx = x_v[pl.ds(i * _L, _L)]
        out_v[pl.ds(i * _L, _L)] = lax.gather(
            t, idx[:, None],
            lax.GatherDimensionNumbers(
                offset_dims=(), collapsed_slice_dims=(0,),
                start_index_map=(0,)),
            (1,),
            mode=lax.GatherScatterMode.PROMISE_IN_BOUNDS,
        )
        return carry

    lax.fori_loop(0, _CH // _L, step, 0)
    pltpu.sync_copy(out_v, out_hbm.at[pl.ds(base, _CH)])


_sc_call = pl.kernel(
    _body,
    mesh=plsc.VectorSubcoreMesh(core_axis_name="c", subcore_axis_name="s"),
    out_type=jax.ShapeDtypeStruct((_N,), jnp.float32),
    scratch_types=[
        pltpu.VMEM((_CH,), jnp.int32),
        pltpu.VMEM((_CH,), jnp.float32),
        pltpu.VMEM((_L,), jnp.float32),
    ],
)


def kernel(x, data_bias):
    xf = x.reshape(-1).astype(jnp.int32)
    tab = jnp.zeros((_L,), jnp.float32).at[:3].set(data_bias.reshape(-1))
    out = _sc_call(xf, tab)
    return out.reshape(16384, 100, 1)


# speedup vs baseline: 1.1195x; 1.1195x over previous
import functools

import jax
import jax.numpy as jnp
from jax import lax
from jax.experimental import pallas as pl
from jax.experimental.pallas import tpu as pltpu
from jax.experimental.pallas import tpu_sc as plsc

_NC = 2
_NS = 16
_NW = _NC * _NS
_L = 16

_N = 16384 * 100
_CH = _N // _NW


def _body(x_hbm, tab_hbm, out_hbm, x_v, out_v, tab_v):
    wid = lax.axis_index("s") * _NC + lax.axis_index("c")
    base = wid * _CH
    pltpu.sync_copy(tab_hbm, tab_v)
    pltpu.sync_copy(x_hbm.at[pl.ds(base, _CH)], x_v)
    v = tab_v[...]
    t = 1.0 - 2.0 / (jnp.exp(2.0 * v) + 1.0)

    @plsc.parallel_loop(0, _CH, _L, unroll=16)
    def step(i):
        idx = x_v[pl.ds(i, _L)]
        out_v[pl.ds(i, _L)] = lax.gather(
            t, idx[:, None],
            lax.GatherDimensionNumbers(
                offset_dims=(), collapsed_slice_dims=(0,),
                start_index_map=(0,)),
            (1,),
            mode=lax.GatherScatterMode.PROMISE_IN_BOUNDS,
        )
    pltpu.sync_copy(out_v, out_hbm.at[pl.ds(base, _CH)])


_sc_call = pl.kernel(
    _body,
    mesh=plsc.VectorSubcoreMesh(core_axis_name="c", subcore_axis_name="s"),
    out_type=jax.ShapeDtypeStruct((_N,), jnp.float32),
    scratch_types=[
        pltpu.VMEM((_CH,), jnp.int32),
        pltpu.VMEM((_CH,), jnp.float32),
        pltpu.VMEM((_L,), jnp.float32),
    ],
)


def kernel(x, data_bias):
    xf = x.reshape(-1).astype(jnp.int32)
    tab = jnp.zeros((_L,), jnp.float32).at[:3].set(data_bias.reshape(-1))
    out = _sc_call(xf, tab)
    return out.reshape(16384, 100, 1)


# speedup vs baseline: 1.8375x; 1.6413x over previous
import functools

import jax
import jax.numpy as jnp
from jax import lax
from jax.experimental import pallas as pl
from jax.experimental.pallas import tpu as pltpu
from jax.experimental.pallas import tpu_sc as plsc

_NC = 2
_NS = 16
_NW = _NC * _NS
_L = 16

_R = 16384
_D = 100
_RW = _R // _NW
_CR = 256
_NCH = _RW // _CR
_OFFS = (0, 16, 32, 48, 64, 80, 84)

_DNUMS = lax.GatherDimensionNumbers(
    offset_dims=(), collapsed_slice_dims=(0,), start_index_map=(0,))


def _body(x_hbm, tab_hbm, out_hbm, x_v, out_v, tab_v):
    wid = lax.axis_index("s") * _NC + lax.axis_index("c")
    base = wid * _RW
    pltpu.sync_copy(tab_hbm, tab_v)
    v = tab_v[...]
    t = 1.0 - 2.0 / (jnp.exp(2.0 * v) + 1.0)

    def chunk(c, carry):
        row0 = base + c * _CR
        pltpu.sync_copy(x_hbm.at[pl.ds(row0, _CR), :], x_v)

        @plsc.parallel_loop(0, _CR, 1, unroll=4)
        def row(r):
            for o in _OFFS:
                idx = x_v[r, pl.ds(o, _L)]
                out_v[r, pl.ds(o, _L)] = lax.gather(
                    t, idx[:, None], _DNUMS, (1,),
                    mode=lax.GatherScatterMode.PROMISE_IN_BOUNDS)

        pltpu.sync_copy(out_v, out_hbm.at[pl.ds(row0, _CR), :])
        return carry

    lax.fori_loop(0, _NCH, chunk, 0)


_sc_call = pl.kernel(
    _body,
    mesh=plsc.VectorSubcoreMesh(core_axis_name="c", subcore_axis_name="s"),
    out_type=jax.ShapeDtypeStruct((_R, _D), jnp.float32),
    scratch_types=[
        pltpu.VMEM((_CR, _D), jnp.int32),
        pltpu.VMEM((_CR, _D), jnp.float32),
        pltpu.VMEM((_L,), jnp.float32),
    ],
)


def kernel(x, data_bias):
    tab = jnp.zeros((_L,), jnp.float32).at[:3].set(data_bias.reshape(-1))
    return _sc_call(x.astype(jnp.int32), tab).reshape(_R, _D, 1)


# speedup vs baseline: 7.7050x; 4.1932x over previous
import functools

import jax
import jax.numpy as jnp
from jax.experimental import pallas as pl
from jax.experimental.pallas import tpu as pltpu

_R = 16384
_D = 100
_BI = 2048
_G = _R // _BI
_BA = _BI // 128


def _body(bias_ref, xt_ref, o_ref):
    t = jnp.tanh(bias_ref[...])
    xb = xt_ref[...]
    t0, t1, t2 = t[0, 0], t[1, 0], t[2, 0]
    sel = jnp.where(xb == 0, t0, jnp.where(xb == 1, t1, t2))
    o_ref[...] = sel.reshape(_D, _BA, 128)


@jax.jit
def kernel(x, data_bias):
    xt = x.astype(jnp.int32).T
    res = pl.pallas_call(
        _body,
        grid=(_G,),
        in_specs=[
            pl.BlockSpec((3, 1), lambda i: (0, 0)),
            pl.BlockSpec((_D, _BI), lambda i: (0, i)),
        ],
        out_specs=pl.BlockSpec((_D, _BA, 128), lambda i: (0, i, 0)),
        out_shape=jax.ShapeDtypeStruct((_D, _R // 128, 128), jnp.float32),
    )(data_bias, xt)
    return jnp.transpose(res, (1, 2, 0)).reshape(_R, _D, 1)
